# Initial kernel scaffold; baseline (speedup 1.0000x reference)
import jax
import jax.numpy as jnp
from jax.experimental import pallas as pl


def kernel(X):
    raise NotImplementedError("write your pallas kernel here")


# speedup vs baseline: 10.1132x; 10.1132x over previous
import jax
import jax.numpy as jnp
from jax.experimental import pallas as pl

_FILL = 1e-15


def _max_layer_kernel(x_ref, o_ref):
    x = x_ref[0]
    M, N = x.shape
    iota = jax.lax.broadcasted_iota(jnp.int32, (M, N), 1)
    rmax = jnp.max(x, axis=1, keepdims=True)
    ismax = x == rmax
    idx = jnp.min(jnp.where(ismax, iota, N), axis=1, keepdims=True)
    onehot = (iota == idx).astype(jnp.float32)
    cnt = jax.lax.dot_general(
        onehot,
        jnp.ones((M, 1), jnp.float32),
        (((0,), (0,)), ((), ())),
        preferred_element_type=jnp.float32,
    )
    keep = cnt > 0.5
    o_ref[0] = jnp.full((M, N), _FILL, dtype=jnp.float32)
    o_ref[0, :N, :] = jnp.where(keep, x[:N, :], jnp.full((N, N), _FILL, jnp.float32))


@jax.jit
def kernel(X):
    B, M, N = X.shape
    return pl.pallas_call(
        _max_layer_kernel,
        grid=(B,),
        in_specs=[pl.BlockSpec((1, M, N), lambda i: (i, 0, 0))],
        out_specs=pl.BlockSpec((1, M, N), lambda i: (i, 0, 0)),
        out_shape=jax.ShapeDtypeStruct((B, M, N), jnp.float32),
    )(X)
